# baseline (speedup 1.0000x reference)
import functools

import jax
import jax.numpy as jnp
import numpy as np
from jax import lax
from jax.experimental import pallas as pl
from jax.experimental.pallas import tpu as pltpu
from jax.experimental.pallas import tpu_sc as plsc

F32 = jnp.float32
I32 = jnp.int32

LAT = 128
N = 10000
NP = 10240
E = 160000
EP = 163840
NB = 400
NBP = 512
EB = 800
EBP = 1024
DEPTH = 4
C = float(1.0 / np.sqrt(2.0))
ACC_N = 10112
PADDST = ACC_N - 1

_NC, _NS = 2, 16
_RPA = ACC_N // _NS
_K = 64
_NCH = EP // (_K * _NC * _NS)


def _erf(s):
    a = jnp.abs(s)
    t = 1.0 / (1.0 + 0.3275911 * a)
    poly = t * (0.254829592 + t * (-0.284496736 + t * (1.421413741
                + t * (-1.453152027 + t * 1.061405429))))
    e = poly * jnp.exp(-a * a)
    return jnp.where(s < 0, e - 1.0, 1.0 - e)


def _gelu(x):
    return 0.5 * x * (1.0 + _erf(x * C))


def _gelu_pre(s):
    return 0.7071067811865476 * s * (1.0 + _erf(s))


def _ln(g, b, x):
    mu = jnp.mean(x, -1, keepdims=True)
    var = jnp.mean((x - mu) ** 2, -1, keepdims=True)
    return (x - mu) / jnp.sqrt(var + 1e-5) * g + b


def _k1_body(nc, mask, w1, b1, w2, b2, w1d, w1s, xo, pdo, pso):
    h = _gelu(jnp.dot(nc[...], w1[...], preferred_element_type=F32) + b1[...])
    x = _gelu(jnp.dot(h, w2[...], preferred_element_type=F32) + b2[...])
    x = x * mask[...]
    xo[...] = x
    pdo[...] = jnp.dot(x, w1d[...], preferred_element_type=F32)
    pso[...] = jnp.dot(x, w1s[...], preferred_element_type=F32)


def _k1(nc, mask, w1, b1, w2, b2, w1d, w1s):
    T = 1024
    full = lambda shp: pl.BlockSpec(shp, lambda i: tuple(0 for _ in shp))
    return pl.pallas_call(
        _k1_body,
        grid=(NP // T,),
        in_specs=[pl.BlockSpec((T, 128), lambda i: (i, 0)),
                  pl.BlockSpec((T, 128), lambda i: (i, 0)),
                  full((128, 64)), full((1, 64)), full((64, 128)), full((1, 128)),
                  full((128, 128)), full((128, 128))],
        out_specs=[pl.BlockSpec((T, 128), lambda i: (i, 0))] * 3,
        out_shape=[jax.ShapeDtypeStruct((NP, 128), F32)] * 3,
    )(nc, mask, w1, b1, w2, b2, w1d, w1s)


def _k2_body(ea, w1, b1, w2, b2, w1e, b1e, lng, lnb, pe0, pe1, pe2, pe3):
    h = _gelu(jnp.dot(ea[...], w1[...], preferred_element_type=F32) + b1[...])
    e = _gelu(jnp.dot(h, w2[...], preferred_element_type=F32) + b2[...])
    outs = (pe0, pe1, pe2, pe3)
    for k in range(DEPTH):
        outs[k][...] = jnp.dot(e, w1e[k], preferred_element_type=F32) + b1e[k]
        if k < DEPTH - 1:
            e = _ln(lng[k], lnb[k], _gelu(2.0 * e))


def _k2(ea, w1, b1, w2, b2, w1e, b1e, lng, lnb):
    T = 2048
    full = lambda shp: pl.BlockSpec(shp, lambda i: tuple(0 for _ in shp))
    return pl.pallas_call(
        _k2_body,
        grid=(EP // T,),
        in_specs=[pl.BlockSpec((T, 8), lambda i: (i, 0)),
                  full((8, 64)), full((1, 64)), full((64, 128)), full((1, 128)),
                  full((4, 128, 128)), full((4, 128)),
                  full((3, 128)), full((3, 128))],
        out_specs=[pl.BlockSpec((T, 128), lambda i: (i, 0))] * 4,
        out_shape=[jax.ShapeDtypeStruct((EP, 128), F32)] * 4,
    )(ea, w1, b1, w2, b2, w1e, b1e, lng, lnb)


def _k3_body(nc, bni, bdst, bsrc, bdstr, bea,
             pw1, pb1, pw2, pb2, ew1, eb1, ew2, eb2,
             gw, gwe, gad, gas, gae, gb, beg, beb, out):
    bni_col = bni[...][:, :1]

    def chunk(i, acc):
        base = pl.multiple_of(i * 1024, 1024)
        m = (bni_col == lax.broadcasted_iota(I32, (NBP, 1024), 1) + i * 1024)
        return acc + jnp.dot(m.astype(F32), nc[pl.ds(base, 1024), :],
                             preferred_element_type=F32)

    ncb = lax.fori_loop(0, NP // 1024, chunk, jnp.zeros((NBP, 128), F32))

    h1 = _gelu(jnp.dot(ncb, pw1[...], preferred_element_type=F32) + pb1[...])
    x = _gelu(jnp.dot(h1, pw2[...], preferred_element_type=F32) + pb2[...])
    h2 = _gelu(jnp.dot(bea[...], ew1[...], preferred_element_type=F32) + eb1[...])
    be = _gelu(jnp.dot(h2, ew2[...], preferred_element_type=F32) + eb2[...])

    dst_col = bdst[...][:, :1]
    src_col = bsrc[...][:, :1]
    sd = (dst_col == lax.broadcasted_iota(I32, (EBP, NBP), 1)).astype(F32)
    ss = (src_col == lax.broadcasted_iota(I32, (EBP, NBP), 1)).astype(F32)
    sdt = (bdstr[...][:1, :] == lax.broadcasted_iota(I32, (NBP, EBP), 0)
           ).astype(F32)

    n_g = gw.shape[0]
    for i in range(n_g):
        h = jnp.dot(x, gw[i], preferred_element_type=F32)
        he = jnp.dot(be, gwe[i], preferred_element_type=F32)
        z = (jnp.dot(sd, jnp.dot(h, gad[...][:, i:i + 1],
                                 preferred_element_type=F32),
                     preferred_element_type=F32)
             + jnp.dot(ss, jnp.dot(h, gas[...][:, i:i + 1],
                                   preferred_element_type=F32),
                       preferred_element_type=F32)
             + jnp.dot(he, gae[...][:, i:i + 1], preferred_element_type=F32))
        z = jnp.where(z > 0, z, 0.2 * z)
        mrow = jnp.max(jnp.where(sd > 0, z, -jnp.inf), axis=0, keepdims=True)
        mrow = jnp.where(mrow > -3e38, mrow, 0.0)
        ez = jnp.exp(z - jnp.sum(sd * mrow, axis=1, keepdims=True))
        srow = jnp.sum(sd * ez, axis=0, keepdims=True)
        alpha = ez / (jnp.sum(sd * srow, axis=1, keepdims=True) + 1e-16)
        hs = jnp.dot(ss, h, preferred_element_type=F32)
        x = jnp.dot(sdt, alpha * hs, preferred_element_type=F32) + gb[i]
        if i < n_g - 1:
            x = jnp.where(x > 0, x, jnp.exp(x) - 1.0)

    rowmask = (lax.broadcasted_iota(I32, (NBP, 1), 0) < NB).astype(F32)
    mean = jnp.sum(x * rowmask, axis=0, keepdims=True) / float(NB)
    out[...] = jnp.broadcast_to(_ln(beg[...], beb[...], mean), (8, 128))


def _k3(nc, bni, bdst, bsrc, bdstr, bea, pw1, pb1, pw2, pb2,
        ew1, eb1, ew2, eb2, gw, gwe, gad, gas, gae, gb, beg, beb):
    full = lambda shp: pl.BlockSpec(shp, lambda: tuple(0 for _ in shp))
    n_g = gw.shape[0]
    return pl.pallas_call(
        _k3_body,
        in_specs=[full((NP, 128)), full((NBP, 128)), full((EBP, 128)),
                  full((EBP, 128)), full((8, EBP)), full((EBP, 8)),
                  full((128, 64)), full((1, 64)), full((64, 128)), full((1, 128)),
                  full((8, 64)), full((1, 64)), full((64, 128)), full((1, 128)),
                  full((n_g, 128, 128)), full((n_g, 128, 128)),
                  full((128, n_g)), full((128, n_g)), full((128, n_g)),
                  full((n_g, 128)), full((1, 128)), full((1, 128))],
        out_specs=full((8, 128)),
        out_shape=jax.ShapeDtypeStruct((8, 128), F32),
    )(nc, bni, bdst, bsrc, bdstr, bea, pw1, pb1, pw2, pb2,
      ew1, eb1, ew2, eb2, gw, gwe, gad, gas, gae, gb, beg, beb)


def _sc_segsum(pd, ps, pe, di, si):
    mesh = plsc.VectorSubcoreMesh(core_axis_name="c", subcore_axis_name="s",
                                  num_cores=_NC, num_subcores=_NS)

    def body(pd_h, ps_h, pe_h, di_h, si_h, out_h,
             pid0, pis0, pid1, pis1, rd0, rs0, pe0, rd1, rs1, pe1,
             acc, si0, si1, sd0, ss0, se0, sd1, ss1, se1, sc0, sc1):
        c = lax.axis_index("c")
        s = lax.axis_index("s")
        base = s * _RPA

        def zrow(r, _):
            for j in range(8):
                rd0[r, pl.ds(j * 16, 16)] = jnp.zeros((16,), F32)
            return 0

        lax.fori_loop(0, _K, zrow, 0)
        for jj in range(_RPA // _K):
            pltpu.sync_copy(rd0, acc.at[pl.ds(base + jj * _K, _K), :])
        pltpu.sync_copy(rd0.at[pl.ds(0, _RPA % _K)],
                        acc.at[pl.ds(base + (_RPA // _K) * _K, _RPA % _K), :])
        plsc.subcore_barrier()

        tid = c * _NS + s
        ebase = tid * _NCH * _K

        pidx = ((pid0, pis0, si0), (pid1, pis1, si1))
        bufs = ((rd0, rs0, pe0, sd0, ss0, se0, sc0),
                (rd1, rs1, pe1, sd1, ss1, se1, sc1))

        def idx_load(p, pb):
            pid, pis, sm = pidx[pb]
            off = ebase + p * 2 * _K
            pltpu.async_copy(di_h.at[pl.ds(off, 2 * _K)], pid, sm)
            pltpu.async_copy(si_h.at[pl.ds(off, 2 * _K)], pis, sm)

        def idx_wait(pb):
            pid, pis, sm = pidx[pb]
            dummy = di_h.at[pl.ds(0, 2 * _K)]
            pltpu.make_async_copy(dummy, pid, sm).wait()
            pltpu.make_async_copy(dummy, pis, sm).wait()

        def load(chi, b, pb, half, wait_sc=True):
            rd, rs, pe, s1, s2, s3, scm = bufs[b]
            pid, pis, _ = pidx[pb]
            pltpu.async_copy(pd_h.at[pid.at[pl.ds(half * _K, _K)]], rd, s1)
            pltpu.async_copy(ps_h.at[pis.at[pl.ds(half * _K, _K)]], rs, s2)
            if wait_sc:
                pltpu.make_async_copy(pd_h.at[pl.ds(0, _K), :], pe,
                                      scm).wait()
            pltpu.async_copy(pe_h.at[pl.ds(ebase + chi * _K, _K), :], pe, s3)

        def consume(b, pb, half, async_sc=True):
            rd, rs, pe, s1, s2, s3, scm = bufs[b]
            pid = pidx[pb][0]
            dummy = pd_h.at[pl.ds(0, _K), :]
            pltpu.make_async_copy(dummy, rd, s1).wait()
            pltpu.make_async_copy(dummy, rs, s2).wait()
            pltpu.make_async_copy(dummy, pe, s3).wait()

            def crow(h, _):
                for rr in range(2):
                    r = 2 * h + rr
                    for j in range(8):
                        sl = pl.ds(j * 16, 16)
                        pe[r, sl] = _gelu_pre(rd[r, sl] + rs[r, sl]
                                              + pe[r, sl])
                return 0

            lax.fori_loop(0, _K // 2, crow, 0)
            tgt = acc.at[pid.at[pl.ds(half * _K, _K)]]
            if async_sc:
                pltpu.async_copy(pe, tgt, scm, add=True)
            else:
                pltpu.sync_copy(pe, tgt, add=True)

        pltpu.sync_copy(di_h.at[pl.ds(ebase, 2 * _K)], pid0)
        pltpu.sync_copy(si_h.at[pl.ds(ebase, 2 * _K)], pis0)
        load(0, 0, 0, 0, wait_sc=False)
        idx_load(1, 1)

        def quad_body(q, first):
            cb = 4 * q
            idx_wait(1)
            load(cb + 1, 1, 0, 1, wait_sc=not first)
            consume(0, 0, 0)
            load(cb + 2, 0, 1, 0)
            consume(1, 0, 1)
            idx_load(2 * q + 2, 0)
            load(cb + 3, 1, 1, 1)
            consume(0, 1, 0)
            idx_wait(0)
            load(cb + 4, 0, 0, 0)
            consume(1, 1, 1)
            idx_load(2 * q + 3, 1)

        quad_body(0, True)
        lax.fori_loop(1, _NCH // 4 - 1,
                      lambda q, _: (quad_body(q, False), 0)[1], 0)
        cb = _NCH - 4
        idx_wait(1)
        load(cb + 1, 1, 0, 1)
        consume(0, 0, 0)
        load(cb + 2, 0, 1, 0)
        consume(1, 0, 1)
        load(cb + 3, 1, 1, 1)
        consume(0, 1, 0, async_sc=False)
        consume(1, 1, 1, async_sc=False)

        plsc.subcore_barrier()
        pltpu.sync_copy(acc.at[pl.ds(base, _RPA), :],
                        out_h.at[c, pl.ds(base, _RPA), :])

    f = pl.kernel(
        body,
        out_type=jax.ShapeDtypeStruct((_NC, NP, 128), F32),
        mesh=mesh,
        scratch_types=[pltpu.VMEM((2 * _K,), I32), pltpu.VMEM((2 * _K,), I32),
                       pltpu.VMEM((2 * _K,), I32), pltpu.VMEM((2 * _K,), I32),
                       pltpu.VMEM((_K, 128), F32), pltpu.VMEM((_K, 128), F32),
                       pltpu.VMEM((_K, 128), F32),
                       pltpu.VMEM((_K, 128), F32), pltpu.VMEM((_K, 128), F32),
                       pltpu.VMEM((_K, 128), F32),
                       pltpu.VMEM_SHARED((ACC_N, 128), F32),
                       pltpu.SemaphoreType.DMA, pltpu.SemaphoreType.DMA,
                       pltpu.SemaphoreType.DMA, pltpu.SemaphoreType.DMA,
                       pltpu.SemaphoreType.DMA, pltpu.SemaphoreType.DMA,
                       pltpu.SemaphoreType.DMA, pltpu.SemaphoreType.DMA,
                       pltpu.SemaphoreType.DMA, pltpu.SemaphoreType.DMA])
    return f(pd, ps, pe, di, si)


def _sc_count(di):
    mesh = plsc.VectorSubcoreMesh(core_axis_name="c", subcore_axis_name="s",
                                  num_cores=_NC, num_subcores=_NS)

    _KB = 256
    _NB = EP // (_KB * _NC * _NS)

    def body(di_h, out_h, id0, id1, ones, acc, sm0, sm1):
        c = lax.axis_index("c")
        s = lax.axis_index("s")
        base = s * _RPA

        def zrow(r, _):
            for j in range(8):
                ones[r, pl.ds(j * 16, 16)] = jnp.zeros((16,), F32)
            return 0

        lax.fori_loop(0, _KB, zrow, 0)
        for jj in range(_RPA // _KB):
            pltpu.sync_copy(ones, acc.at[pl.ds(base + jj * _KB, _KB), :])
        pltpu.sync_copy(ones.at[pl.ds(0, _RPA % _KB)],
                        acc.at[pl.ds(base + (_RPA // _KB) * _KB,
                                     _RPA % _KB), :])

        def orow(r, _):
            for j in range(8):
                ones[r, pl.ds(j * 16, 16)] = jnp.full((16,), 1.0, F32)
            return 0

        lax.fori_loop(0, _KB, orow, 0)
        plsc.subcore_barrier()

        tid = c * _NS + s
        ebase = tid * _NB * _KB
        bufs = ((id0, sm0), (id1, sm1))

        def idx_load(bt, b):
            pltpu.async_copy(di_h.at[pl.ds(ebase + bt * _KB, _KB)],
                             bufs[b][0], bufs[b][1])

        def scat(b):
            idb, sm = bufs[b]
            pltpu.make_async_copy(di_h.at[pl.ds(0, _KB)], idb, sm).wait()
            pltpu.sync_copy(ones, acc.at[idb], add=True)

        idx_load(0, 0)
        idx_load(1, 1)

        def dbatch(g, _):
            scat(0)
            idx_load(2 * g + 2, 0)
            scat(1)
            idx_load(2 * g + 3, 1)
            return 0

        lax.fori_loop(0, _NB // 2 - 1, dbatch, 0)
        scat(0)
        scat(1)
        plsc.subcore_barrier()
        pltpu.sync_copy(acc.at[pl.ds(base, _RPA), :],
                        out_h.at[c, pl.ds(base, _RPA), :])

    f = pl.kernel(
        body,
        out_type=jax.ShapeDtypeStruct((_NC, NP, 128), F32),
        mesh=mesh,
        scratch_types=[pltpu.VMEM((_KB,), I32), pltpu.VMEM((_KB,), I32),
                       pltpu.VMEM((_KB, 128), F32),
                       pltpu.VMEM_SHARED((ACC_N, 128), F32),
                       pltpu.SemaphoreType.DMA, pltpu.SemaphoreType.DMA])
    return f(di)


def _k4_body(x, sh0, sh1, c0, c1, bnd, w2e, b2e, wn1a, wn1b, wn1c, bn1,
             wn2, bn2, nng, nnb, w1d, w1s, xo, pdo, pso):
    cnt = (c0[...] + c1[...])[:, :1]
    maxc = jnp.maximum(cnt, 1.0)
    flag = jnp.minimum(cnt, 1.0)
    aggh = (sh0[...] + sh1[...]) / maxc
    agg = jnp.dot(aggh, w2e[...], preferred_element_type=F32) + flag * b2e[...]
    bt = jnp.dot(bnd[...][:1, :], wn1c[...], preferred_element_type=F32) + bn1[...]
    u = jnp.dot(_gelu(jnp.dot(agg, wn1a[...], preferred_element_type=F32)
                      + jnp.dot(x[...], wn1b[...], preferred_element_type=F32)
                      + bt),
                wn2[...], preferred_element_type=F32) + bn2[...]
    xn = _ln(nng[...], nnb[...], _gelu(x[...] + u))
    xo[...] = xn
    pdo[...] = jnp.dot(xn, w1d[...], preferred_element_type=F32)
    pso[...] = jnp.dot(xn, w1s[...], preferred_element_type=F32)


def _k4(x, sh0, sh1, c0, c1, bnd, w2e, b2e, wn1a, wn1b, wn1c, bn1,
        wn2, bn2, nng, nnb, w1d, w1s):
    T = 1024
    full = lambda shp: pl.BlockSpec(shp, lambda i: tuple(0 for _ in shp))
    row = lambda w: pl.BlockSpec((T, w), lambda i: (i, 0))
    return pl.pallas_call(
        _k4_body,
        grid=(NP // T,),
        in_specs=[row(128), row(128), row(128), row(128), row(128),
                  full((8, 128)),
                  full((128, 128)), full((1, 128)),
                  full((128, 128)), full((128, 128)), full((128, 128)),
                  full((1, 128)), full((128, 128)), full((1, 128)),
                  full((1, 128)), full((1, 128)),
                  full((128, 128)), full((128, 128))],
        out_specs=[row(128)] * 3,
        out_shape=[jax.ShapeDtypeStruct((NP, 128), F32)] * 3,
    )(x, sh0, sh1, c0, c1, bnd, w2e, b2e, wn1a, wn1b, wn1c, bn1,
      wn2, bn2, nng, nnb, w1d, w1s)


def _k5_body(x, sh0, sh1, c0, c1, bnd, w2e, b2e, wn1a, wn1b, wn1c, bn1,
             wn2, bn2, nng, nnb, eng, enb, dw1, db1, dw2, db2,
             edw1, edb1, edw2, edb2, yo):
    cnt = (c0[...] + c1[...])[:, :1]
    maxc = jnp.maximum(cnt, 1.0)
    flag = jnp.minimum(cnt, 1.0)
    aggh = (sh0[...] + sh1[...]) / maxc
    agg = jnp.dot(aggh, w2e[...], preferred_element_type=F32) + flag * b2e[...]
    bt = jnp.dot(bnd[...][:1, :], wn1c[...], preferred_element_type=F32) + bn1[...]
    u = jnp.dot(_gelu(jnp.dot(agg, wn1a[...], preferred_element_type=F32)
                      + jnp.dot(x[...], wn1b[...], preferred_element_type=F32)
                      + bt),
                wn2[...], preferred_element_type=F32) + bn2[...]
    x4 = _ln(nng[...], nnb[...], _gelu(x[...] + u))
    xi = _ln(eng[...], enb[...], x4)
    y1 = jnp.dot(_gelu(jnp.dot(xi, dw1[...], preferred_element_type=F32)
                       + db1[...]), dw2[...], preferred_element_type=F32) + db2[...]
    y2 = jnp.dot(_gelu(jnp.dot(x4, edw1[...], preferred_element_type=F32)
                       + edb1[...]), edw2[...], preferred_element_type=F32) + edb2[...]
    yo[...] = y1 + y2


def _k5(x, sh0, sh1, c0, c1, bnd, w2e, b2e, wn1a, wn1b, wn1c, bn1,
        wn2, bn2, nng, nnb, eng, enb, dw1, db1, dw2, db2,
        edw1, edb1, edw2, edb2):
    T = 1024
    full = lambda shp: pl.BlockSpec(shp, lambda i: tuple(0 for _ in shp))
    row = lambda w: pl.BlockSpec((T, w), lambda i: (i, 0))
    return pl.pallas_call(
        _k5_body,
        grid=(NP // T,),
        in_specs=[row(128), row(128), row(128), row(128), row(128),
                  full((8, 128)),
                  full((128, 128)), full((1, 128)),
                  full((128, 128)), full((128, 128)), full((128, 128)),
                  full((1, 128)), full((128, 128)), full((1, 128)),
                  full((1, 128)), full((1, 128)), full((1, 128)), full((1, 128)),
                  full((128, 64)), full((1, 64)), full((64, 8)), full((1, 8)),
                  full((128, 64)), full((1, 64)), full((64, 8)), full((1, 8))],
        out_specs=row(8),
        out_shape=jax.ShapeDtypeStruct((NP, 8), F32),
    )(x, sh0, sh1, c0, c1, bnd, w2e, b2e, wn1a, wn1b, wn1c, bn1,
      wn2, bn2, nng, nnb, eng, enb, dw1, db1, dw2, db2,
      edw1, edb1, edw2, edb2)


def kernel(nodes, grid, edge_index, edge_attr, boundary_edge_index,
           boundary_edge_attr, boundary_node_index, boundary_node_mask,
           batch_size, image_size, params):
    p = params

    nc = jnp.concatenate([nodes, grid], -1)
    ncp = jnp.zeros((NP, 128), F32).at[:N, :18].set(nc)
    maskp = jnp.zeros((NP, 1), F32).at[:N].set(boundary_node_mask)
    mask_b = jnp.broadcast_to(maskp, (NP, 128))

    src = edge_index[0]
    dst = edge_index[1]
    dstp = jnp.full((EP,), PADDST, I32).at[:E].set(dst)
    srcp = jnp.zeros((EP,), I32).at[:E].set(src)
    eap = jnp.zeros((EP, 8), F32).at[:E, :7].set(edge_attr)

    bni = jnp.zeros((NBP,), I32).at[:NB].set(boundary_node_index)
    bni_b = jnp.broadcast_to(bni[:, None], (NBP, 128))
    bdst = jnp.full((EBP,), NBP - 1, I32).at[:EB].set(boundary_edge_index[1])
    bsrc = jnp.zeros((EBP,), I32).at[:EB].set(boundary_edge_index[0])
    bdst_b = jnp.broadcast_to(bdst[:, None], (EBP, 128))
    bsrc_b = jnp.broadcast_to(bsrc[:, None], (EBP, 128))
    beap = jnp.zeros((EBP, 8), F32).at[:EB, :5].set(boundary_edge_attr)

    def lin2(q, din_pad=None):
        w1 = q["l1"]["w"]
        if din_pad is not None and w1.shape[0] < din_pad:
            w1 = jnp.zeros((din_pad, w1.shape[1]), F32).at[:w1.shape[0]].set(w1)
        return (w1, q["l1"]["b"][None, :], q["l2"]["w"], q["l2"]["b"][None, :])

    pw1, pb1, pw2, pb2 = lin2(p["external_projector"], 128)
    ew1, eb1, ew2, eb2 = lin2(p["external_edge_projector"], 8)
    bpw1, bpb1, bpw2, bpb2 = lin2(p["boundary_projector"], 128)
    bew1, beb1, bew2, beb2 = lin2(p["boundary_edge_projector"], 8)

    blocks = p["external_blocks"]
    w1d = [C * b["edge_func"]["l1"]["w"][:128] for b in blocks]
    w1s = [C * b["edge_func"]["l1"]["w"][128:256] for b in blocks]
    w1e = jnp.stack([C * b["edge_func"]["l1"]["w"][256:] for b in blocks])
    b1e = jnp.stack([C * b["edge_func"]["l1"]["b"] for b in blocks])
    elng = jnp.stack([blocks[k]["edge_norm"]["g"] for k in range(DEPTH - 1)])
    elnb = jnp.stack([blocks[k]["edge_norm"]["b"] for k in range(DEPTH - 1)])

    gat = p["gat"]
    gw = jnp.stack([q["w"] for q in gat])
    gwe = jnp.stack([q["we"] for q in gat])
    gad = jnp.stack([q["a_dst"] for q in gat], axis=1)
    gas = jnp.stack([q["a_src"] for q in gat], axis=1)
    gae = jnp.stack([q["a_e"] for q in gat], axis=1)
    gb = jnp.stack([q["b"] for q in gat])
    bdst_row = jnp.broadcast_to(bdst[None, :], (8, EBP))

    x, pd, ps = _k1(ncp, mask_b, pw1, pb1, pw2, pb2, w1d[0], w1s[0])
    pes = _k2(eap, ew1, eb1, ew2, eb2, w1e, b1e, elng, elnb)
    bnd = _k3(ncp, bni_b, bdst_b, bsrc_b, bdst_row, beap,
              bpw1, bpb1, bpw2, bpb2, bew1, beb1, bew2, beb2,
              gw, gwe, gad, gas, gae, gb,
              p["be_norm"]["g"][None, :], p["be_norm"]["b"][None, :])

    cp = _sc_count(dstp)
    for k in range(DEPTH):
        blk = blocks[k]
        sh = _sc_segsum(pd, ps, pes[k], dstp, srcp)
        args = (x, sh[0], sh[1], cp[0], cp[1], bnd,
                blk["edge_func"]["l2"]["w"], blk["edge_func"]["l2"]["b"][None, :],
                blk["node_func"]["l1"]["w"][:128],
                blk["node_func"]["l1"]["w"][128:256],
                blk["node_func"]["l1"]["w"][256:],
                blk["node_func"]["l1"]["b"][None, :],
                blk["node_func"]["l2"]["w"], blk["node_func"]["l2"]["b"][None, :],
                blk["node_norm"]["g"][None, :], blk["node_norm"]["b"][None, :])
        if k < DEPTH - 1:
            x, pd, ps = _k4(*args, w1d[k + 1], w1s[k + 1])
        else:
            dw1, db1, dw2, db2 = lin2(p["decoder"])
            edw1, edb1, edw2, edb2 = lin2(p["external_decoder"])
            dw2 = jnp.zeros((64, 8), F32).at[:, :1].set(dw2)
            db2 = jnp.zeros((1, 8), F32).at[:, :1].set(db2)
            edw2 = jnp.zeros((64, 8), F32).at[:, :1].set(edw2)
            edb2 = jnp.zeros((1, 8), F32).at[:, :1].set(edb2)
            y = _k5(*args, p["external_norm"]["g"][None, :],
                    p["external_norm"]["b"][None, :],
                    dw1, db1, dw2, db2, edw1, edb1, edw2, edb2)

    return y[:N, :1]


# speedup vs baseline: 1.0169x; 1.0169x over previous
import functools

import jax
import jax.numpy as jnp
import numpy as np
from jax import lax
from jax.experimental import pallas as pl
from jax.experimental.pallas import tpu as pltpu
from jax.experimental.pallas import tpu_sc as plsc

F32 = jnp.float32
I32 = jnp.int32

LAT = 128
N = 10000
NP = 10240
E = 160000
EP = 163840
NB = 400
NBP = 512
EB = 800
EBP = 1024
DEPTH = 4
C = float(1.0 / np.sqrt(2.0))
ACC_N = 10112
PADDST = ACC_N - 1

_NC, _NS = 2, 16
_RPA = ACC_N // _NS
_K = 64
_NCH = EP // (_K * _NC * _NS)


def _erf(s):
    a = jnp.abs(s)
    t = 1.0 / (1.0 + 0.3275911 * a)
    poly = t * (0.254829592 + t * (-0.284496736 + t * (1.421413741
                + t * (-1.453152027 + t * 1.061405429))))
    e = poly * jnp.exp(-a * a)
    return jnp.where(s < 0, e - 1.0, 1.0 - e)


def _gelu(x):
    return 0.5 * x * (1.0 + _erf(x * C))


def _gelu_pre(s):
    return 0.7071067811865476 * s * (1.0 + _erf(s))


def _ln(g, b, x):
    mu = jnp.mean(x, -1, keepdims=True)
    var = jnp.mean((x - mu) ** 2, -1, keepdims=True)
    return (x - mu) / jnp.sqrt(var + 1e-5) * g + b


def _k1_body(nc, mask, w1, b1, w2, b2, w1d, w1s, xo, pdo, pso):
    h = _gelu(jnp.dot(nc[...], w1[...], preferred_element_type=F32) + b1[...])
    x = _gelu(jnp.dot(h, w2[...], preferred_element_type=F32) + b2[...])
    x = x * mask[...]
    xo[...] = x
    pdo[...] = jnp.dot(x, w1d[...], preferred_element_type=F32)
    pso[...] = jnp.dot(x, w1s[...], preferred_element_type=F32)


def _k1(nc, mask, w1, b1, w2, b2, w1d, w1s):
    T = 1024
    full = lambda shp: pl.BlockSpec(shp, lambda i: tuple(0 for _ in shp))
    return pl.pallas_call(
        _k1_body,
        grid=(NP // T,),
        in_specs=[pl.BlockSpec((T, 128), lambda i: (i, 0)),
                  pl.BlockSpec((T, 128), lambda i: (i, 0)),
                  full((128, 64)), full((1, 64)), full((64, 128)), full((1, 128)),
                  full((128, 128)), full((128, 128))],
        out_specs=[pl.BlockSpec((T, 128), lambda i: (i, 0))] * 3,
        out_shape=[jax.ShapeDtypeStruct((NP, 128), F32)] * 3,
    )(nc, mask, w1, b1, w2, b2, w1d, w1s)


def _k2_body(ea, w1, b1, w2, b2, w1e, b1e, lng, lnb, pe0, pe1, pe2, pe3):
    h = _gelu(jnp.dot(ea[...], w1[...], preferred_element_type=F32) + b1[...])
    e = _gelu(jnp.dot(h, w2[...], preferred_element_type=F32) + b2[...])
    outs = (pe0, pe1, pe2, pe3)
    for k in range(DEPTH):
        outs[k][...] = jnp.dot(e, w1e[k], preferred_element_type=F32) + b1e[k]
        if k < DEPTH - 1:
            e = _ln(lng[k], lnb[k], _gelu(2.0 * e))


def _k2(ea, w1, b1, w2, b2, w1e, b1e, lng, lnb):
    T = 2048
    full = lambda shp: pl.BlockSpec(shp, lambda i: tuple(0 for _ in shp))
    return pl.pallas_call(
        _k2_body,
        grid=(EP // T,),
        in_specs=[pl.BlockSpec((T, 8), lambda i: (i, 0)),
                  full((8, 64)), full((1, 64)), full((64, 128)), full((1, 128)),
                  full((4, 128, 128)), full((4, 128)),
                  full((3, 128)), full((3, 128))],
        out_specs=[pl.BlockSpec((T, 128), lambda i: (i, 0))] * 4,
        out_shape=[jax.ShapeDtypeStruct((EP, 128), F32)] * 4,
    )(ea, w1, b1, w2, b2, w1e, b1e, lng, lnb)


def _k3_body(nc, bni, bdst, bsrc, bdstr, bea,
             pw1, pb1, pw2, pb2, ew1, eb1, ew2, eb2,
             gw, gwe, gad, gas, gae, gb, beg, beb, out):
    bni_col = bni[...][:, :1]

    def chunk(i, acc):
        base = pl.multiple_of(i * 1024, 1024)
        m = (bni_col == lax.broadcasted_iota(I32, (NBP, 1024), 1) + i * 1024)
        return acc + jnp.dot(m.astype(F32), nc[pl.ds(base, 1024), :],
                             preferred_element_type=F32)

    ncb = lax.fori_loop(0, NP // 1024, chunk, jnp.zeros((NBP, 128), F32))

    h1 = _gelu(jnp.dot(ncb, pw1[...], preferred_element_type=F32) + pb1[...])
    x = _gelu(jnp.dot(h1, pw2[...], preferred_element_type=F32) + pb2[...])
    h2 = _gelu(jnp.dot(bea[...], ew1[...], preferred_element_type=F32) + eb1[...])
    be = _gelu(jnp.dot(h2, ew2[...], preferred_element_type=F32) + eb2[...])

    dst_col = bdst[...][:, :1]
    src_col = bsrc[...][:, :1]
    sd = (dst_col == lax.broadcasted_iota(I32, (EBP, NBP), 1)).astype(F32)
    ss = (src_col == lax.broadcasted_iota(I32, (EBP, NBP), 1)).astype(F32)
    sdt = (bdstr[...][:1, :] == lax.broadcasted_iota(I32, (NBP, EBP), 0)
           ).astype(F32)

    n_g = gw.shape[0]
    for i in range(n_g):
        h = jnp.dot(x, gw[i], preferred_element_type=F32)
        he = jnp.dot(be, gwe[i], preferred_element_type=F32)
        z = (jnp.dot(sd, jnp.dot(h, gad[...][:, i:i + 1],
                                 preferred_element_type=F32),
                     preferred_element_type=F32)
             + jnp.dot(ss, jnp.dot(h, gas[...][:, i:i + 1],
                                   preferred_element_type=F32),
                       preferred_element_type=F32)
             + jnp.dot(he, gae[...][:, i:i + 1], preferred_element_type=F32))
        z = jnp.where(z > 0, z, 0.2 * z)
        mrow = jnp.max(jnp.where(sd > 0, z, -jnp.inf), axis=0, keepdims=True)
        mrow = jnp.where(mrow > -3e38, mrow, 0.0)
        ez = jnp.exp(z - jnp.sum(sd * mrow, axis=1, keepdims=True))
        srow = jnp.sum(sd * ez, axis=0, keepdims=True)
        alpha = ez / (jnp.sum(sd * srow, axis=1, keepdims=True) + 1e-16)
        hs = jnp.dot(ss, h, preferred_element_type=F32)
        x = jnp.dot(sdt, alpha * hs, preferred_element_type=F32) + gb[i]
        if i < n_g - 1:
            x = jnp.where(x > 0, x, jnp.exp(x) - 1.0)

    rowmask = (lax.broadcasted_iota(I32, (NBP, 1), 0) < NB).astype(F32)
    mean = jnp.sum(x * rowmask, axis=0, keepdims=True) / float(NB)
    out[...] = jnp.broadcast_to(_ln(beg[...], beb[...], mean), (8, 128))


def _k3(nc, bni, bdst, bsrc, bdstr, bea, pw1, pb1, pw2, pb2,
        ew1, eb1, ew2, eb2, gw, gwe, gad, gas, gae, gb, beg, beb):
    full = lambda shp: pl.BlockSpec(shp, lambda: tuple(0 for _ in shp))
    n_g = gw.shape[0]
    return pl.pallas_call(
        _k3_body,
        in_specs=[full((NP, 128)), full((NBP, 128)), full((EBP, 128)),
                  full((EBP, 128)), full((8, EBP)), full((EBP, 8)),
                  full((128, 64)), full((1, 64)), full((64, 128)), full((1, 128)),
                  full((8, 64)), full((1, 64)), full((64, 128)), full((1, 128)),
                  full((n_g, 128, 128)), full((n_g, 128, 128)),
                  full((128, n_g)), full((128, n_g)), full((128, n_g)),
                  full((n_g, 128)), full((1, 128)), full((1, 128))],
        out_specs=full((8, 128)),
        out_shape=jax.ShapeDtypeStruct((8, 128), F32),
    )(nc, bni, bdst, bsrc, bdstr, bea, pw1, pb1, pw2, pb2,
      ew1, eb1, ew2, eb2, gw, gwe, gad, gas, gae, gb, beg, beb)


def _sc_segsum(pd, ps, pe, di, si):
    mesh = plsc.VectorSubcoreMesh(core_axis_name="c", subcore_axis_name="s",
                                  num_cores=_NC, num_subcores=_NS)

    def body(pd_h, ps_h, pe_h, di_h, si_h, out_h,
             pid0, pis0, pid1, pis1, rd0, rs0, pe0, rd1, rs1, pe1,
             acc, si0, si1, sd0, ss0, se0, sd1, ss1, se1, sc0, sc1):
        c = lax.axis_index("c")
        s = lax.axis_index("s")
        base = s * _RPA

        def zrow(r, _):
            for j in range(8):
                rd0[r, pl.ds(j * 16, 16)] = jnp.zeros((16,), F32)
            return 0

        lax.fori_loop(0, _K, zrow, 0)
        for jj in range(_RPA // _K):
            pltpu.sync_copy(rd0, acc.at[pl.ds(base + jj * _K, _K), :])
        pltpu.sync_copy(rd0.at[pl.ds(0, _RPA % _K)],
                        acc.at[pl.ds(base + (_RPA // _K) * _K, _RPA % _K), :])
        plsc.subcore_barrier()

        tid = c * _NS + s
        ebase = tid * _NCH * _K

        pidx = ((pid0, pis0, si0), (pid1, pis1, si1))
        bufs = ((rd0, rs0, pe0, sd0, ss0, se0, sc0),
                (rd1, rs1, pe1, sd1, ss1, se1, sc1))

        def idx_load(p, pb):
            pid, pis, sm = pidx[pb]
            off = ebase + p * 2 * _K
            pltpu.async_copy(di_h.at[pl.ds(off, 2 * _K)], pid, sm)
            pltpu.async_copy(si_h.at[pl.ds(off, 2 * _K)], pis, sm)

        def idx_wait(pb):
            pid, pis, sm = pidx[pb]
            dummy = di_h.at[pl.ds(0, 2 * _K)]
            pltpu.make_async_copy(dummy, pid, sm).wait()
            pltpu.make_async_copy(dummy, pis, sm).wait()

        def load(chi, b, pb, half, wait_sc=True):
            rd, rs, pe, s1, s2, s3, scm = bufs[b]
            pid, pis, _ = pidx[pb]
            pltpu.async_copy(pd_h.at[pid.at[pl.ds(half * _K, _K)]], rd, s1)
            pltpu.async_copy(ps_h.at[pis.at[pl.ds(half * _K, _K)]], rs, s2)
            if wait_sc:
                pltpu.make_async_copy(pd_h.at[pl.ds(0, _K), :], pe,
                                      scm).wait()
            pltpu.async_copy(pe_h.at[pl.ds(ebase + chi * _K, _K), :], pe, s3)

        def consume(b, pb, half, async_sc=True):
            rd, rs, pe, s1, s2, s3, scm = bufs[b]
            pid = pidx[pb][0]
            dummy = pd_h.at[pl.ds(0, _K), :]
            pltpu.make_async_copy(dummy, rd, s1).wait()
            pltpu.make_async_copy(dummy, rs, s2).wait()
            pltpu.make_async_copy(dummy, pe, s3).wait()

            def crow(r, _):
                for j in range(8):
                    sl = pl.ds(j * 16, 16)
                    pe[r, sl] = _gelu_pre(rd[r, sl] + rs[r, sl] + pe[r, sl])
                return 0

            lax.fori_loop(0, _K, crow, 0)
            tgt = acc.at[pid.at[pl.ds(half * _K, _K)]]
            if async_sc:
                pltpu.async_copy(pe, tgt, scm, add=True)
            else:
                pltpu.sync_copy(pe, tgt, add=True)

        pltpu.sync_copy(di_h.at[pl.ds(ebase, 2 * _K)], pid0)
        pltpu.sync_copy(si_h.at[pl.ds(ebase, 2 * _K)], pis0)
        load(0, 0, 0, 0, wait_sc=False)
        idx_load(1, 1)

        def quad_body(q, first):
            cb = 4 * q
            idx_wait(1)
            load(cb + 1, 1, 0, 1, wait_sc=not first)
            consume(0, 0, 0)
            load(cb + 2, 0, 1, 0)
            consume(1, 0, 1)
            idx_load(2 * q + 2, 0)
            load(cb + 3, 1, 1, 1)
            consume(0, 1, 0)
            idx_wait(0)
            load(cb + 4, 0, 0, 0)
            consume(1, 1, 1)
            idx_load(2 * q + 3, 1)

        quad_body(0, True)
        lax.fori_loop(1, _NCH // 4 - 1,
                      lambda q, _: (quad_body(q, False), 0)[1], 0)
        cb = _NCH - 4
        idx_wait(1)
        load(cb + 1, 1, 0, 1)
        consume(0, 0, 0)
        load(cb + 2, 0, 1, 0)
        consume(1, 0, 1)
        load(cb + 3, 1, 1, 1)
        consume(0, 1, 0, async_sc=False)
        consume(1, 1, 1, async_sc=False)

        plsc.subcore_barrier()
        pltpu.sync_copy(acc.at[pl.ds(base, _RPA), :],
                        out_h.at[c, pl.ds(base, _RPA), :])

    f = pl.kernel(
        body,
        out_type=jax.ShapeDtypeStruct((_NC, NP, 128), F32),
        mesh=mesh,
        scratch_types=[pltpu.VMEM((2 * _K,), I32), pltpu.VMEM((2 * _K,), I32),
                       pltpu.VMEM((2 * _K,), I32), pltpu.VMEM((2 * _K,), I32),
                       pltpu.VMEM((_K, 128), F32), pltpu.VMEM((_K, 128), F32),
                       pltpu.VMEM((_K, 128), F32),
                       pltpu.VMEM((_K, 128), F32), pltpu.VMEM((_K, 128), F32),
                       pltpu.VMEM((_K, 128), F32),
                       pltpu.VMEM_SHARED((ACC_N, 128), F32),
                       pltpu.SemaphoreType.DMA, pltpu.SemaphoreType.DMA,
                       pltpu.SemaphoreType.DMA, pltpu.SemaphoreType.DMA,
                       pltpu.SemaphoreType.DMA, pltpu.SemaphoreType.DMA,
                       pltpu.SemaphoreType.DMA, pltpu.SemaphoreType.DMA,
                       pltpu.SemaphoreType.DMA, pltpu.SemaphoreType.DMA])
    return f(pd, ps, pe, di, si)


def _sc_count(di):
    mesh = plsc.VectorSubcoreMesh(core_axis_name="c", subcore_axis_name="s",
                                  num_cores=_NC, num_subcores=_NS)

    _KB = 256
    _NB = EP // (_KB * _NC * _NS)

    def body(di_h, out_h, id0, id1, ones, acc, sm0, sm1):
        c = lax.axis_index("c")
        s = lax.axis_index("s")
        base = s * _RPA

        def zrow(r, _):
            for j in range(8):
                ones[r, pl.ds(j * 16, 16)] = jnp.zeros((16,), F32)
            return 0

        lax.fori_loop(0, _KB, zrow, 0)
        for jj in range(_RPA // _KB):
            pltpu.sync_copy(ones, acc.at[pl.ds(base + jj * _KB, _KB), :])
        pltpu.sync_copy(ones.at[pl.ds(0, _RPA % _KB)],
                        acc.at[pl.ds(base + (_RPA // _KB) * _KB,
                                     _RPA % _KB), :])

        def orow(r, _):
            for j in range(8):
                ones[r, pl.ds(j * 16, 16)] = jnp.full((16,), 1.0, F32)
            return 0

        lax.fori_loop(0, _KB, orow, 0)
        plsc.subcore_barrier()

        tid = c * _NS + s
        ebase = tid * _NB * _KB
        bufs = ((id0, sm0), (id1, sm1))

        def idx_load(bt, b):
            pltpu.async_copy(di_h.at[pl.ds(ebase + bt * _KB, _KB)],
                             bufs[b][0], bufs[b][1])

        def scat(b):
            idb, sm = bufs[b]
            pltpu.make_async_copy(di_h.at[pl.ds(0, _KB)], idb, sm).wait()
            pltpu.sync_copy(ones, acc.at[idb], add=True)

        idx_load(0, 0)
        idx_load(1, 1)

        def dbatch(g, _):
            scat(0)
            idx_load(2 * g + 2, 0)
            scat(1)
            idx_load(2 * g + 3, 1)
            return 0

        lax.fori_loop(0, _NB // 2 - 1, dbatch, 0)
        scat(0)
        scat(1)
        plsc.subcore_barrier()
        pltpu.sync_copy(acc.at[pl.ds(base, _RPA), :],
                        out_h.at[c, pl.ds(base, _RPA), :])

    f = pl.kernel(
        body,
        out_type=jax.ShapeDtypeStruct((_NC, NP, 128), F32),
        mesh=mesh,
        scratch_types=[pltpu.VMEM((_KB,), I32), pltpu.VMEM((_KB,), I32),
                       pltpu.VMEM((_KB, 128), F32),
                       pltpu.VMEM_SHARED((ACC_N, 128), F32),
                       pltpu.SemaphoreType.DMA, pltpu.SemaphoreType.DMA])
    return f(di)


def _k4_body(x, sh0, sh1, c0, c1, bnd, w2e, b2e, wn1a, wn1b, wn1c, bn1,
             wn2, bn2, nng, nnb, w1d, w1s, xo, pdo, pso):
    cnt = (c0[...] + c1[...])[:, :1]
    maxc = jnp.maximum(cnt, 1.0)
    flag = jnp.minimum(cnt, 1.0)
    aggh = (sh0[...] + sh1[...]) / maxc
    agg = jnp.dot(aggh, w2e[...], preferred_element_type=F32) + flag * b2e[...]
    bt = jnp.dot(bnd[...][:1, :], wn1c[...], preferred_element_type=F32) + bn1[...]
    u = jnp.dot(_gelu(jnp.dot(agg, wn1a[...], preferred_element_type=F32)
                      + jnp.dot(x[...], wn1b[...], preferred_element_type=F32)
                      + bt),
                wn2[...], preferred_element_type=F32) + bn2[...]
    xn = _ln(nng[...], nnb[...], _gelu(x[...] + u))
    xo[...] = xn
    pdo[...] = jnp.dot(xn, w1d[...], preferred_element_type=F32)
    pso[...] = jnp.dot(xn, w1s[...], preferred_element_type=F32)


def _k4(x, sh0, sh1, c0, c1, bnd, w2e, b2e, wn1a, wn1b, wn1c, bn1,
        wn2, bn2, nng, nnb, w1d, w1s):
    T = 1024
    full = lambda shp: pl.BlockSpec(shp, lambda i: tuple(0 for _ in shp))
    row = lambda w: pl.BlockSpec((T, w), lambda i: (i, 0))
    return pl.pallas_call(
        _k4_body,
        grid=(NP // T,),
        in_specs=[row(128), row(128), row(128), row(128), row(128),
                  full((8, 128)),
                  full((128, 128)), full((1, 128)),
                  full((128, 128)), full((128, 128)), full((128, 128)),
                  full((1, 128)), full((128, 128)), full((1, 128)),
                  full((1, 128)), full((1, 128)),
                  full((128, 128)), full((128, 128))],
        out_specs=[row(128)] * 3,
        out_shape=[jax.ShapeDtypeStruct((NP, 128), F32)] * 3,
    )(x, sh0, sh1, c0, c1, bnd, w2e, b2e, wn1a, wn1b, wn1c, bn1,
      wn2, bn2, nng, nnb, w1d, w1s)


def _k5_body(x, sh0, sh1, c0, c1, bnd, w2e, b2e, wn1a, wn1b, wn1c, bn1,
             wn2, bn2, nng, nnb, eng, enb, dw1, db1, dw2, db2,
             edw1, edb1, edw2, edb2, yo):
    cnt = (c0[...] + c1[...])[:, :1]
    maxc = jnp.maximum(cnt, 1.0)
    flag = jnp.minimum(cnt, 1.0)
    aggh = (sh0[...] + sh1[...]) / maxc
    agg = jnp.dot(aggh, w2e[...], preferred_element_type=F32) + flag * b2e[...]
    bt = jnp.dot(bnd[...][:1, :], wn1c[...], preferred_element_type=F32) + bn1[...]
    u = jnp.dot(_gelu(jnp.dot(agg, wn1a[...], preferred_element_type=F32)
                      + jnp.dot(x[...], wn1b[...], preferred_element_type=F32)
                      + bt),
                wn2[...], preferred_element_type=F32) + bn2[...]
    x4 = _ln(nng[...], nnb[...], _gelu(x[...] + u))
    xi = _ln(eng[...], enb[...], x4)
    y1 = jnp.dot(_gelu(jnp.dot(xi, dw1[...], preferred_element_type=F32)
                       + db1[...]), dw2[...], preferred_element_type=F32) + db2[...]
    y2 = jnp.dot(_gelu(jnp.dot(x4, edw1[...], preferred_element_type=F32)
                       + edb1[...]), edw2[...], preferred_element_type=F32) + edb2[...]
    yo[...] = y1 + y2


def _k5(x, sh0, sh1, c0, c1, bnd, w2e, b2e, wn1a, wn1b, wn1c, bn1,
        wn2, bn2, nng, nnb, eng, enb, dw1, db1, dw2, db2,
        edw1, edb1, edw2, edb2):
    T = 1024
    full = lambda shp: pl.BlockSpec(shp, lambda i: tuple(0 for _ in shp))
    row = lambda w: pl.BlockSpec((T, w), lambda i: (i, 0))
    return pl.pallas_call(
        _k5_body,
        grid=(NP // T,),
        in_specs=[row(128), row(128), row(128), row(128), row(128),
                  full((8, 128)),
                  full((128, 128)), full((1, 128)),
                  full((128, 128)), full((128, 128)), full((128, 128)),
                  full((1, 128)), full((128, 128)), full((1, 128)),
                  full((1, 128)), full((1, 128)), full((1, 128)), full((1, 128)),
                  full((128, 64)), full((1, 64)), full((64, 8)), full((1, 8)),
                  full((128, 64)), full((1, 64)), full((64, 8)), full((1, 8))],
        out_specs=row(8),
        out_shape=jax.ShapeDtypeStruct((NP, 8), F32),
    )(x, sh0, sh1, c0, c1, bnd, w2e, b2e, wn1a, wn1b, wn1c, bn1,
      wn2, bn2, nng, nnb, eng, enb, dw1, db1, dw2, db2,
      edw1, edb1, edw2, edb2)


def kernel(nodes, grid, edge_index, edge_attr, boundary_edge_index,
           boundary_edge_attr, boundary_node_index, boundary_node_mask,
           batch_size, image_size, params):
    p = params

    nc = jnp.concatenate([nodes, grid], -1)
    ncp = jnp.zeros((NP, 128), F32).at[:N, :18].set(nc)
    maskp = jnp.zeros((NP, 1), F32).at[:N].set(boundary_node_mask)
    mask_b = jnp.broadcast_to(maskp, (NP, 128))

    src = edge_index[0]
    dst = edge_index[1]
    dstp = jnp.full((EP,), PADDST, I32).at[:E].set(dst)
    srcp = jnp.zeros((EP,), I32).at[:E].set(src)
    eap = jnp.zeros((EP, 8), F32).at[:E, :7].set(edge_attr)

    bni = jnp.zeros((NBP,), I32).at[:NB].set(boundary_node_index)
    bni_b = jnp.broadcast_to(bni[:, None], (NBP, 128))
    bdst = jnp.full((EBP,), NBP - 1, I32).at[:EB].set(boundary_edge_index[1])
    bsrc = jnp.zeros((EBP,), I32).at[:EB].set(boundary_edge_index[0])
    bdst_b = jnp.broadcast_to(bdst[:, None], (EBP, 128))
    bsrc_b = jnp.broadcast_to(bsrc[:, None], (EBP, 128))
    beap = jnp.zeros((EBP, 8), F32).at[:EB, :5].set(boundary_edge_attr)

    def lin2(q, din_pad=None):
        w1 = q["l1"]["w"]
        if din_pad is not None and w1.shape[0] < din_pad:
            w1 = jnp.zeros((din_pad, w1.shape[1]), F32).at[:w1.shape[0]].set(w1)
        return (w1, q["l1"]["b"][None, :], q["l2"]["w"], q["l2"]["b"][None, :])

    pw1, pb1, pw2, pb2 = lin2(p["external_projector"], 128)
    ew1, eb1, ew2, eb2 = lin2(p["external_edge_projector"], 8)
    bpw1, bpb1, bpw2, bpb2 = lin2(p["boundary_projector"], 128)
    bew1, beb1, bew2, beb2 = lin2(p["boundary_edge_projector"], 8)

    blocks = p["external_blocks"]
    w1d = [C * b["edge_func"]["l1"]["w"][:128] for b in blocks]
    w1s = [C * b["edge_func"]["l1"]["w"][128:256] for b in blocks]
    w1e = jnp.stack([C * b["edge_func"]["l1"]["w"][256:] for b in blocks])
    b1e = jnp.stack([C * b["edge_func"]["l1"]["b"] for b in blocks])
    elng = jnp.stack([blocks[k]["edge_norm"]["g"] for k in range(DEPTH - 1)])
    elnb = jnp.stack([blocks[k]["edge_norm"]["b"] for k in range(DEPTH - 1)])

    gat = p["gat"]
    gw = jnp.stack([q["w"] for q in gat])
    gwe = jnp.stack([q["we"] for q in gat])
    gad = jnp.stack([q["a_dst"] for q in gat], axis=1)
    gas = jnp.stack([q["a_src"] for q in gat], axis=1)
    gae = jnp.stack([q["a_e"] for q in gat], axis=1)
    gb = jnp.stack([q["b"] for q in gat])
    bdst_row = jnp.broadcast_to(bdst[None, :], (8, EBP))

    x, pd, ps = _k1(ncp, mask_b, pw1, pb1, pw2, pb2, w1d[0], w1s[0])
    pes = _k2(eap, ew1, eb1, ew2, eb2, w1e, b1e, elng, elnb)
    bnd = _k3(ncp, bni_b, bdst_b, bsrc_b, bdst_row, beap,
              bpw1, bpb1, bpw2, bpb2, bew1, beb1, bew2, beb2,
              gw, gwe, gad, gas, gae, gb,
              p["be_norm"]["g"][None, :], p["be_norm"]["b"][None, :])

    cp = _sc_count(dstp)
    for k in range(DEPTH):
        blk = blocks[k]
        sh = _sc_segsum(pd, ps, pes[k], dstp, srcp)
        args = (x, sh[0], sh[1], cp[0], cp[1], bnd,
                blk["edge_func"]["l2"]["w"], blk["edge_func"]["l2"]["b"][None, :],
                blk["node_func"]["l1"]["w"][:128],
                blk["node_func"]["l1"]["w"][128:256],
                blk["node_func"]["l1"]["w"][256:],
                blk["node_func"]["l1"]["b"][None, :],
                blk["node_func"]["l2"]["w"], blk["node_func"]["l2"]["b"][None, :],
                blk["node_norm"]["g"][None, :], blk["node_norm"]["b"][None, :])
        if k < DEPTH - 1:
            x, pd, ps = _k4(*args, w1d[k + 1], w1s[k + 1])
        else:
            dw1, db1, dw2, db2 = lin2(p["decoder"])
            edw1, edb1, edw2, edb2 = lin2(p["external_decoder"])
            dw2 = jnp.zeros((64, 8), F32).at[:, :1].set(dw2)
            db2 = jnp.zeros((1, 8), F32).at[:, :1].set(db2)
            edw2 = jnp.zeros((64, 8), F32).at[:, :1].set(edw2)
            edb2 = jnp.zeros((1, 8), F32).at[:, :1].set(edb2)
            y = _k5(*args, p["external_norm"]["g"][None, :],
                    p["external_norm"]["b"][None, :],
                    dw1, db1, dw2, db2, edw1, edb1, edw2, edb2)

    return y[:N, :1]
